# Initial kernel scaffold; baseline (speedup 1.0000x reference)
import jax
import jax.numpy as jnp
from jax.experimental import pallas as pl


def kernel(output, target):
    raise NotImplementedError("write your pallas kernel here")


# speedup vs baseline: 29.3805x; 29.3805x over previous
import functools

import jax
import jax.numpy as jnp
from jax import lax
from jax.experimental import pallas as pl
from jax.experimental.pallas import tpu as pltpu
from jax.experimental.pallas import tpu_sc as plsc

_N = 4194304
_K = _N // 2
_R = 4096
_C = 1024
_BR = 128
_NW = 32
_PER_W = _N // _NW
_CH = 16384
_U = 8
_L = 16


def _dense_body(o_ref, g_ref, keys_ref, s1_ref, acc_ref):
    i = pl.program_id(0)
    t = o_ref[...] * g_ref[...]
    u = lax.bitcast_convert_type(t, jnp.int32)
    keys_ref[...] = u ^ ((u >> 31) & jnp.int32(0x7FFFFFFF))
    bsum = jnp.sum(jnp.log(jnp.exp(-t) + 1.0))

    @pl.when(i == 0)
    def _():
        acc_ref[0, 0] = jnp.float32(0.0)

    acc_ref[0, 0] += bsum

    @pl.when(i == pl.num_programs(0) - 1)
    def _():
        s1_ref[0, 0] = acc_ref[0, 0]


_dense_pass = pl.pallas_call(
    _dense_body,
    grid=(_R // _BR,),
    in_specs=[
        pl.BlockSpec((_BR, _C), lambda i: (i, 0)),
        pl.BlockSpec((_BR, _C), lambda i: (i, 0)),
    ],
    out_specs=[
        pl.BlockSpec((_BR, _C), lambda i: (i, 0)),
        pl.BlockSpec(memory_space=pltpu.SMEM),
    ],
    out_shape=[
        jax.ShapeDtypeStruct((_R, _C), jnp.int32),
        jax.ShapeDtypeStruct((1, 1), jnp.float32),
    ],
    scratch_shapes=[pltpu.SMEM((1, 1), jnp.float32)],
)


def _tail_body(keys_ref, tau_ref, s2_ref, cnt_ref, facc_ref, cacc_ref):
    i = pl.program_id(0)
    key = keys_ref[...]
    tau = tau_ref[0, 0]
    m = key < tau
    u = key ^ ((key >> 31) & jnp.int32(0x7FFFFFFF))
    t = lax.bitcast_convert_type(u, jnp.float32)
    sp = jnp.log(jnp.exp(t) + 1.0)
    bsum = jnp.sum(jnp.where(m, sp, jnp.float32(0.0)))
    bcnt = jnp.sum(m.astype(jnp.int32))

    @pl.when(i == 0)
    def _():
        facc_ref[0, 0] = jnp.float32(0.0)
        cacc_ref[0, 0] = jnp.int32(0)

    facc_ref[0, 0] += bsum
    cacc_ref[0, 0] += bcnt

    @pl.when(i == pl.num_programs(0) - 1)
    def _():
        s2_ref[0, 0] = facc_ref[0, 0]
        cnt_ref[0, 0] = cacc_ref[0, 0]


_tail_pass = pl.pallas_call(
    _tail_body,
    grid=(_R // _BR,),
    in_specs=[
        pl.BlockSpec((_BR, _C), lambda i: (i, 0)),
        pl.BlockSpec(memory_space=pltpu.SMEM),
    ],
    out_specs=[
        pl.BlockSpec(memory_space=pltpu.SMEM),
        pl.BlockSpec(memory_space=pltpu.SMEM),
    ],
    out_shape=[
        jax.ShapeDtypeStruct((1, 1), jnp.float32),
        jax.ShapeDtypeStruct((1, 1), jnp.int32),
    ],
    scratch_shapes=[
        pltpu.SMEM((1, 1), jnp.float32),
        pltpu.SMEM((1, 1), jnp.int32),
    ],
)


def _make_sc_pass(shift, nbits, match):
    nb = 1 << nbits
    mesh = plsc.VectorSubcoreMesh(core_axis_name="c", subcore_axis_name="s")

    def body(keys_hbm, pref_hbm, zeros_hbm, hist_out, buf_v, hist_v,
             merged_v, pref_v, shared):
        cid = lax.axis_index("c")
        sid = lax.axis_index("s")
        wid = cid * 16 + sid
        base = wid * _PER_W

        pltpu.sync_copy(zeros_hbm, hist_v)
        pltpu.sync_copy(pref_hbm, pref_v)
        pref = pref_v[...]
        lane_off = lax.iota(jnp.int32, _L) * nb
        ones = jnp.ones((_L,), jnp.int32)

        def chunk_body(ci, carry):
            pltpu.sync_copy(keys_hbm.at[pl.ds(base + ci * _CH, _CH)], buf_v)

            def vec_body(vi, c2):
                off = vi * (_L * _U)
                for s in range(_U):
                    kv = buf_v[pl.ds(off + s * _L, _L)]
                    uk = kv ^ jnp.int32(-2147483648)
                    bucket = lax.shift_right_logical(uk, shift) & jnp.int32(nb - 1)
                    idx = bucket + lane_off
                    if match:
                        pm = lax.shift_right_logical(uk, shift + nbits) == pref
                        plsc.addupdate_scatter(hist_v, [idx], ones, mask=pm)
                    else:
                        plsc.addupdate_scatter(hist_v, [idx], ones)
                return c2

            lax.fori_loop(0, _CH // (_L * _U), vec_body, 0)
            return carry

        lax.fori_loop(0, _PER_W // _CH, chunk_body, 0)

        def merge16(src_v):
            def mbody(c, carry):
                acc = src_v[pl.ds(c * _L, _L)]
                for r in range(1, 16):
                    acc = acc + src_v[pl.ds(r * nb + c * _L, _L)]
                merged_v[pl.ds(c * _L, _L)] = acc
                return carry

            lax.fori_loop(0, nb // _L, mbody, 0)

        merge16(hist_v)

        pltpu.sync_copy(merged_v, shared.at[pl.ds(sid * nb, nb)])
        plsc.subcore_barrier()

        @pl.when(sid == 0)
        def _():
            pltpu.sync_copy(shared, hist_v)
            merge16(hist_v)
            pltpu.sync_copy(merged_v, hist_out.at[cid])

    return functools.partial(
        pl.kernel,
        mesh=mesh,
        compiler_params=pltpu.CompilerParams(needs_layout_passes=False),
        out_type=jax.ShapeDtypeStruct((2, nb), jnp.int32),
        scratch_types=[
            pltpu.VMEM((_CH,), jnp.int32),
            pltpu.VMEM((16 * nb,), jnp.int32),
            pltpu.VMEM((nb,), jnp.int32),
            pltpu.VMEM((_L,), jnp.int32),
            pltpu.VMEM_SHARED((16 * nb,), jnp.int32),
        ],
    )(body)


_sc_pass1 = _make_sc_pass(20, 12, False)
_sc_pass2 = _make_sc_pass(8, 12, True)
_sc_pass3 = _make_sc_pass(0, 8, True)


def _pick(hist2, kk):
    h = hist2[0] + hist2[1]
    cum = jnp.cumsum(h)
    b = jnp.argmax(cum >= kk).astype(jnp.int32)
    below = cum[b] - h[b]
    return b, below


def kernel(output, target):
    o2 = output.reshape(_R, _C)
    g2 = target.reshape(_R, _C)
    keys2, s1 = _dense_pass(o2, g2)
    keys = keys2.reshape(-1)

    z12 = jnp.zeros((16 * 4096,), jnp.int32)
    z8 = jnp.zeros((16 * 256,), jnp.int32)
    p0 = jnp.zeros((16,), jnp.int32)

    h1 = _sc_pass1(keys, p0, z12)
    b1, below1 = _pick(h1, jnp.int32(_K))
    kk2 = jnp.int32(_K) - below1

    h2 = _sc_pass2(keys, jnp.full((16,), b1, jnp.int32), z12)
    b2, below2 = _pick(h2, kk2)
    kk3 = kk2 - below2

    h3 = _sc_pass3(keys, jnp.full((16,), (b1 << 12) | b2, jnp.int32), z8)
    b3, _ = _pick(h3, kk3)

    u_tau = (b1 << 20) | (b2 << 8) | b3
    tau_key = u_tau ^ jnp.int32(-2147483648)

    s2, cnt = _tail_pass(keys2, tau_key.reshape(1, 1))

    uu = tau_key ^ ((tau_key >> 31) & jnp.int32(0x7FFFFFFF))
    tau_f = lax.bitcast_convert_type(uu, jnp.float32)
    sp_tau = jnp.log(jnp.exp(tau_f) + 1.0)

    total = (s1[0, 0] + s2[0, 0]
             + (jnp.float32(_K) - cnt[0, 0].astype(jnp.float32)) * sp_tau)
    return total / jnp.float32(_N)


# speedup vs baseline: 60.7078x; 2.0663x over previous
import functools

import jax
import jax.numpy as jnp
from jax import lax
from jax.experimental import pallas as pl
from jax.experimental.pallas import tpu as pltpu
from jax.experimental.pallas import tpu_sc as plsc

_N = 4194304
_K = _N // 2
_R = 4096
_C = 1024
_BR = 128
_NW = 32
_PER_W = _N // _NW
_CH = 16384
_U = 8
_L = 16


def _dense_body(o_ref, g_ref, keys_ref, s1_ref, acc_ref):
    i = pl.program_id(0)
    t = o_ref[...] * g_ref[...]
    u = lax.bitcast_convert_type(t, jnp.int32)
    keys_ref[...] = u ^ ((u >> 31) & jnp.int32(0x7FFFFFFF))
    bsum = jnp.sum(jnp.log(jnp.exp(-t) + 1.0))

    @pl.when(i == 0)
    def _():
        acc_ref[0, 0] = jnp.float32(0.0)

    acc_ref[0, 0] += bsum

    @pl.when(i == pl.num_programs(0) - 1)
    def _():
        s1_ref[0, 0] = acc_ref[0, 0]


_dense_pass = pl.pallas_call(
    _dense_body,
    grid=(_R // _BR,),
    in_specs=[
        pl.BlockSpec((_BR, _C), lambda i: (i, 0)),
        pl.BlockSpec((_BR, _C), lambda i: (i, 0)),
    ],
    out_specs=[
        pl.BlockSpec((_BR, _C), lambda i: (i, 0)),
        pl.BlockSpec(memory_space=pltpu.SMEM),
    ],
    out_shape=[
        jax.ShapeDtypeStruct((_R, _C), jnp.int32),
        jax.ShapeDtypeStruct((1, 1), jnp.float32),
    ],
    scratch_shapes=[pltpu.SMEM((1, 1), jnp.float32)],
)


def _tail_body(keys_ref, tau_ref, s2_ref, cnt_ref, facc_ref, cacc_ref):
    i = pl.program_id(0)
    key = keys_ref[...]
    tau = tau_ref[0, 0]
    m = key < tau
    u = key ^ ((key >> 31) & jnp.int32(0x7FFFFFFF))
    t = lax.bitcast_convert_type(u, jnp.float32)
    sp = jnp.log(jnp.exp(t) + 1.0)
    bsum = jnp.sum(jnp.where(m, sp, jnp.float32(0.0)))
    bcnt = jnp.sum(m.astype(jnp.int32))

    @pl.when(i == 0)
    def _():
        facc_ref[0, 0] = jnp.float32(0.0)
        cacc_ref[0, 0] = jnp.int32(0)

    facc_ref[0, 0] += bsum
    cacc_ref[0, 0] += bcnt

    @pl.when(i == pl.num_programs(0) - 1)
    def _():
        s2_ref[0, 0] = facc_ref[0, 0]
        cnt_ref[0, 0] = cacc_ref[0, 0]


_tail_pass = pl.pallas_call(
    _tail_body,
    grid=(_R // _BR,),
    in_specs=[
        pl.BlockSpec((_BR, _C), lambda i: (i, 0)),
        pl.BlockSpec(memory_space=pltpu.SMEM),
    ],
    out_specs=[
        pl.BlockSpec(memory_space=pltpu.SMEM),
        pl.BlockSpec(memory_space=pltpu.SMEM),
    ],
    out_shape=[
        jax.ShapeDtypeStruct((1, 1), jnp.float32),
        jax.ShapeDtypeStruct((1, 1), jnp.int32),
    ],
    scratch_shapes=[
        pltpu.SMEM((1, 1), jnp.float32),
        pltpu.SMEM((1, 1), jnp.int32),
    ],
)


def _make_sc_pass(shift, nbits, match):
    nb = 1 << nbits
    mesh = plsc.VectorSubcoreMesh(core_axis_name="c", subcore_axis_name="s")

    def body(keys_hbm, pref_hbm, zeros_hbm, hist_out, buf_v, hist_v,
             merged_v, pref_v, shared):
        cid = lax.axis_index("c")
        sid = lax.axis_index("s")
        wid = cid * 16 + sid
        base = wid * _PER_W

        pltpu.sync_copy(zeros_hbm, hist_v)
        pltpu.sync_copy(pref_hbm, pref_v)
        pref = pref_v[...]
        lane_off = lax.iota(jnp.int32, _L) * nb
        ones = jnp.ones((_L,), jnp.int32)

        def chunk_body(ci, carry):
            pltpu.sync_copy(keys_hbm.at[pl.ds(base + ci * _CH, _CH)], buf_v)

            @functools.partial(plsc.parallel_loop, 0, _CH // _L, unroll=_U)
            def _(vi):
                kv = buf_v[pl.ds(vi * _L, _L)]
                uk = kv ^ jnp.int32(-2147483648)
                bucket = lax.shift_right_logical(uk, shift) & jnp.int32(nb - 1)
                idx = bucket + lane_off
                if match:
                    pm = lax.shift_right_logical(uk, shift + nbits) == pref
                    plsc.addupdate_scatter(hist_v, [idx], ones, mask=pm)
                else:
                    plsc.addupdate_scatter(hist_v, [idx], ones)

            return carry

        lax.fori_loop(0, _PER_W // _CH, chunk_body, 0)

        def merge16(src_v):
            @functools.partial(plsc.parallel_loop, 0, nb // _L, unroll=4)
            def _(c):
                acc = src_v[pl.ds(c * _L, _L)]
                for r in range(1, 16):
                    acc = acc + src_v[pl.ds(r * nb + c * _L, _L)]
                merged_v[pl.ds(c * _L, _L)] = acc

        merge16(hist_v)

        pltpu.sync_copy(merged_v, shared.at[pl.ds(sid * nb, nb)])
        plsc.subcore_barrier()

        @pl.when(sid == 0)
        def _():
            pltpu.sync_copy(shared, hist_v)
            merge16(hist_v)
            pltpu.sync_copy(merged_v, hist_out.at[cid])

    return functools.partial(
        pl.kernel,
        mesh=mesh,
        compiler_params=pltpu.CompilerParams(needs_layout_passes=False),
        out_type=jax.ShapeDtypeStruct((2, nb), jnp.int32),
        scratch_types=[
            pltpu.VMEM((_CH,), jnp.int32),
            pltpu.VMEM((16 * nb,), jnp.int32),
            pltpu.VMEM((nb,), jnp.int32),
            pltpu.VMEM((_L,), jnp.int32),
            pltpu.VMEM_SHARED((16 * nb,), jnp.int32),
        ],
    )(body)


_sc_pass1 = _make_sc_pass(20, 12, False)
_sc_pass2 = _make_sc_pass(8, 12, True)
_sc_pass3 = _make_sc_pass(0, 8, True)


def _pick(hist2, kk):
    h = hist2[0] + hist2[1]
    cum = jnp.cumsum(h)
    b = jnp.argmax(cum >= kk).astype(jnp.int32)
    below = cum[b] - h[b]
    return b, below


def kernel(output, target):
    o2 = output.reshape(_R, _C)
    g2 = target.reshape(_R, _C)
    keys2, s1 = _dense_pass(o2, g2)
    keys = keys2.reshape(-1)

    z12 = jnp.zeros((16 * 4096,), jnp.int32)
    z8 = jnp.zeros((16 * 256,), jnp.int32)
    p0 = jnp.zeros((16,), jnp.int32)

    h1 = _sc_pass1(keys, p0, z12)
    b1, below1 = _pick(h1, jnp.int32(_K))
    kk2 = jnp.int32(_K) - below1

    h2 = _sc_pass2(keys, jnp.full((16,), b1, jnp.int32), z12)
    b2, below2 = _pick(h2, kk2)
    kk3 = kk2 - below2

    h3 = _sc_pass3(keys, jnp.full((16,), (b1 << 12) | b2, jnp.int32), z8)
    b3, _ = _pick(h3, kk3)

    u_tau = (b1 << 20) | (b2 << 8) | b3
    tau_key = u_tau ^ jnp.int32(-2147483648)

    s2, cnt = _tail_pass(keys2, tau_key.reshape(1, 1))

    uu = tau_key ^ ((tau_key >> 31) & jnp.int32(0x7FFFFFFF))
    tau_f = lax.bitcast_convert_type(uu, jnp.float32)
    sp_tau = jnp.log(jnp.exp(tau_f) + 1.0)

    total = (s1[0, 0] + s2[0, 0]
             + (jnp.float32(_K) - cnt[0, 0].astype(jnp.float32)) * sp_tau)
    return total / jnp.float32(_N)


# speedup vs baseline: 64.0464x; 1.0550x over previous
import functools

import jax
import jax.numpy as jnp
from jax import lax
from jax.experimental import pallas as pl
from jax.experimental.pallas import tpu as pltpu
from jax.experimental.pallas import tpu_sc as plsc

_N = 4194304
_K = _N // 2
_R = 4096
_C = 1024
_BR = 128
_NW = 32
_PER_W = _N // _NW
_CH = 16384
_U = 8
_L = 16


_TB = _N // 32


def _tail_body(o_ref, g_ref, tau_ref, s_ref, cnt_ref, facc_ref, cacc_ref):
    i = pl.program_id(0)
    t = o_ref[...] * g_ref[...]
    u = lax.bitcast_convert_type(t, jnp.int32)
    key = u ^ ((u >> 31) & jnp.int32(0x7FFFFFFF))
    m = key < tau_ref[0, 0]
    val = jnp.log(jnp.exp(-t) + 1.0) + jnp.where(
        m, jnp.log(jnp.exp(t) + 1.0), jnp.float32(0.0))
    bsum = jnp.sum(val)
    bcnt = jnp.sum(m.astype(jnp.int32))

    @pl.when(i == 0)
    def _():
        facc_ref[0, 0] = jnp.float32(0.0)
        cacc_ref[0, 0] = jnp.int32(0)

    facc_ref[0, 0] += bsum
    cacc_ref[0, 0] += bcnt

    @pl.when(i == pl.num_programs(0) - 1)
    def _():
        s_ref[0, 0] = facc_ref[0, 0]
        cnt_ref[0, 0] = cacc_ref[0, 0]


_tail_pass = pl.pallas_call(
    _tail_body,
    grid=(_N // _TB,),
    in_specs=[
        pl.BlockSpec((_TB,), lambda i: (i,)),
        pl.BlockSpec((_TB,), lambda i: (i,)),
        pl.BlockSpec(memory_space=pltpu.SMEM),
    ],
    out_specs=[
        pl.BlockSpec(memory_space=pltpu.SMEM),
        pl.BlockSpec(memory_space=pltpu.SMEM),
    ],
    out_shape=[
        jax.ShapeDtypeStruct((1, 1), jnp.float32),
        jax.ShapeDtypeStruct((1, 1), jnp.int32),
    ],
    scratch_shapes=[
        pltpu.SMEM((1, 1), jnp.float32),
        pltpu.SMEM((1, 1), jnp.int32),
    ],
)


def _make_sc_pass(shift, nbits, match):
    nb = 1 << nbits
    mesh = plsc.VectorSubcoreMesh(core_axis_name="c", subcore_axis_name="s")

    def body(o_hbm, g_hbm, pref_hbm, zeros_hbm, hist_out, obuf_v, gbuf_v,
             hist_v, merged_v, pref_v, shared):
        cid = lax.axis_index("c")
        sid = lax.axis_index("s")
        wid = cid * 16 + sid
        base = wid * _PER_W

        pltpu.sync_copy(zeros_hbm, hist_v)
        pltpu.sync_copy(pref_hbm, pref_v)
        pref = pref_v[...]
        lane_off = lax.iota(jnp.int32, _L) * nb
        ones = jnp.ones((_L,), jnp.int32)

        def chunk_body(ci, carry):
            pltpu.sync_copy(o_hbm.at[pl.ds(base + ci * _CH, _CH)], obuf_v)
            pltpu.sync_copy(g_hbm.at[pl.ds(base + ci * _CH, _CH)], gbuf_v)

            @functools.partial(plsc.parallel_loop, 0, _CH // _L, unroll=_U)
            def _(vi):
                t = obuf_v[pl.ds(vi * _L, _L)] * gbuf_v[pl.ds(vi * _L, _L)]
                u = lax.bitcast_convert_type(t, jnp.int32)
                uk = (u ^ ((u >> 31) & jnp.int32(0x7FFFFFFF))
                      ) ^ jnp.int32(-2147483648)
                bucket = lax.shift_right_logical(uk, shift) & jnp.int32(nb - 1)
                idx = bucket + lane_off
                if match:
                    pm = lax.shift_right_logical(uk, shift + nbits) == pref
                    plsc.addupdate_scatter(hist_v, [idx], ones, mask=pm)
                else:
                    plsc.addupdate_scatter(hist_v, [idx], ones)

            return carry

        lax.fori_loop(0, _PER_W // _CH, chunk_body, 0)

        def merge16(src_v):
            @functools.partial(plsc.parallel_loop, 0, nb // _L, unroll=4)
            def _(c):
                acc = src_v[pl.ds(c * _L, _L)]
                for r in range(1, 16):
                    acc = acc + src_v[pl.ds(r * nb + c * _L, _L)]
                merged_v[pl.ds(c * _L, _L)] = acc

        merge16(hist_v)

        pltpu.sync_copy(merged_v, shared.at[pl.ds(sid * nb, nb)])
        plsc.subcore_barrier()

        @pl.when(sid == 0)
        def _():
            pltpu.sync_copy(shared, hist_v)
            merge16(hist_v)
            pltpu.sync_copy(merged_v, hist_out.at[cid])

    return functools.partial(
        pl.kernel,
        mesh=mesh,
        compiler_params=pltpu.CompilerParams(needs_layout_passes=False),
        out_type=jax.ShapeDtypeStruct((2, nb), jnp.int32),
        scratch_types=[
            pltpu.VMEM((_CH,), jnp.float32),
            pltpu.VMEM((_CH,), jnp.float32),
            pltpu.VMEM((16 * nb,), jnp.int32),
            pltpu.VMEM((nb,), jnp.int32),
            pltpu.VMEM((_L,), jnp.int32),
            pltpu.VMEM_SHARED((16 * nb,), jnp.int32),
        ],
    )(body)


_sc_pass1 = _make_sc_pass(20, 12, False)
_sc_pass2 = _make_sc_pass(8, 12, True)


def _pick(hist2, kk):
    h = hist2[0] + hist2[1]
    cum = jnp.cumsum(h)
    b = jnp.argmax(cum >= kk).astype(jnp.int32)
    below = cum[b] - h[b]
    return b, below


def kernel(output, target):
    z12 = jnp.zeros((16 * 4096,), jnp.int32)

    h1 = _sc_pass1(output, target, jnp.zeros((16,), jnp.int32), z12)
    b1, below1 = _pick(h1, jnp.int32(_K))
    kk2 = jnp.int32(_K) - below1

    h2 = _sc_pass2(output, target, jnp.full((16,), b1, jnp.int32), z12)
    b2, _ = _pick(h2, kk2)

    u_tau = (b1 << 20) | (b2 << 8)
    tau_key = u_tau ^ jnp.int32(-2147483648)

    s, cnt = _tail_pass(output, target, tau_key.reshape(1, 1))

    uu = tau_key ^ ((tau_key >> 31) & jnp.int32(0x7FFFFFFF))
    tau_f = lax.bitcast_convert_type(uu, jnp.float32)
    sp_tau = jnp.log(jnp.exp(tau_f) + 1.0)

    total = s[0, 0] + (jnp.float32(_K) - cnt[0, 0].astype(jnp.float32)) * sp_tau
    return total / jnp.float32(_N)


# speedup vs baseline: 67.9344x; 1.0607x over previous
import functools

import jax
import jax.numpy as jnp
from jax import lax
from jax.experimental import pallas as pl
from jax.experimental.pallas import tpu as pltpu
from jax.experimental.pallas import tpu_sc as plsc

_N = 4194304
_K = _N // 2
_R = 4096
_C = 1024
_BR = 128
_NW = 32
_PER_W = _N // _NW
_CH = 8192
_U = 8
_L = 16


_TB = _N // 32


def _tail_body(o_ref, g_ref, tau_ref, s_ref, cnt_ref, facc_ref, cacc_ref):
    i = pl.program_id(0)
    t = o_ref[...] * g_ref[...]
    u = lax.bitcast_convert_type(t, jnp.int32)
    key = u ^ ((u >> 31) & jnp.int32(0x7FFFFFFF))
    m = key < tau_ref[0, 0]
    val = jnp.log(jnp.exp(-t) + 1.0) + jnp.where(
        m, jnp.log(jnp.exp(t) + 1.0), jnp.float32(0.0))
    bsum = jnp.sum(val)
    bcnt = jnp.sum(m.astype(jnp.int32))

    @pl.when(i == 0)
    def _():
        facc_ref[0, 0] = jnp.float32(0.0)
        cacc_ref[0, 0] = jnp.int32(0)

    facc_ref[0, 0] += bsum
    cacc_ref[0, 0] += bcnt

    @pl.when(i == pl.num_programs(0) - 1)
    def _():
        s_ref[0, 0] = facc_ref[0, 0]
        cnt_ref[0, 0] = cacc_ref[0, 0]


_tail_pass = pl.pallas_call(
    _tail_body,
    grid=(_N // _TB,),
    in_specs=[
        pl.BlockSpec((_TB,), lambda i: (i,)),
        pl.BlockSpec((_TB,), lambda i: (i,)),
        pl.BlockSpec(memory_space=pltpu.SMEM),
    ],
    out_specs=[
        pl.BlockSpec(memory_space=pltpu.SMEM),
        pl.BlockSpec(memory_space=pltpu.SMEM),
    ],
    out_shape=[
        jax.ShapeDtypeStruct((1, 1), jnp.float32),
        jax.ShapeDtypeStruct((1, 1), jnp.int32),
    ],
    scratch_shapes=[
        pltpu.SMEM((1, 1), jnp.float32),
        pltpu.SMEM((1, 1), jnp.int32),
    ],
)


def _make_sc_pass(shift, nbits, match):
    nb = 1 << nbits
    mesh = plsc.VectorSubcoreMesh(core_axis_name="c", subcore_axis_name="s")

    def body(o_hbm, g_hbm, pref_hbm, zeros_hbm, hist_out, o_a, g_a, o_b, g_b,
             hist_v, merged_v, pref_v, shared, sem_a, sem_b):
        cid = lax.axis_index("c")
        sid = lax.axis_index("s")
        wid = cid * 16 + sid
        base = wid * _PER_W

        pltpu.sync_copy(zeros_hbm, hist_v)
        pltpu.sync_copy(pref_hbm, pref_v)
        pref = pref_v[...]
        lane_off = lax.iota(jnp.int32, _L) * nb
        ones = jnp.ones((_L,), jnp.int32)
        nch = _PER_W // _CH

        def start(ci, ob, gb, sem):
            pltpu.async_copy(o_hbm.at[pl.ds(base + ci * _CH, _CH)], ob, sem)
            pltpu.async_copy(g_hbm.at[pl.ds(base + ci * _CH, _CH)], gb, sem)

        def drain(ob, gb, sem):
            pltpu.make_async_copy(o_hbm.at[pl.ds(0, _CH)], ob, sem).wait()
            pltpu.make_async_copy(o_hbm.at[pl.ds(0, _CH)], gb, sem).wait()

        def process(ob, gb):
            @functools.partial(plsc.parallel_loop, 0, _CH // _L, unroll=_U)
            def _(vi):
                t = ob[pl.ds(vi * _L, _L)] * gb[pl.ds(vi * _L, _L)]
                u = lax.bitcast_convert_type(t, jnp.int32)
                uk = (u ^ ((u >> 31) & jnp.int32(0x7FFFFFFF))
                      ) ^ jnp.int32(-2147483648)
                bucket = lax.shift_right_logical(uk, shift) & jnp.int32(nb - 1)
                idx = bucket + lane_off
                if match:
                    pm = lax.shift_right_logical(uk, shift + nbits) == pref
                    plsc.addupdate_scatter(hist_v, [idx], ones, mask=pm)
                else:
                    plsc.addupdate_scatter(hist_v, [idx], ones)

        start(0, o_a, g_a, sem_a)

        def pair_body(p, carry):
            c0 = 2 * p
            start(c0 + 1, o_b, g_b, sem_b)
            drain(o_a, g_a, sem_a)
            process(o_a, g_a)

            @pl.when(c0 + 2 < nch)
            def _():
                start(c0 + 2, o_a, g_a, sem_a)

            drain(o_b, g_b, sem_b)
            process(o_b, g_b)
            return carry

        lax.fori_loop(0, nch // 2, pair_body, 0)

        def merge16(src_v):
            @functools.partial(plsc.parallel_loop, 0, nb // _L, unroll=4)
            def _(c):
                acc = src_v[pl.ds(c * _L, _L)]
                for r in range(1, 16):
                    acc = acc + src_v[pl.ds(r * nb + c * _L, _L)]
                merged_v[pl.ds(c * _L, _L)] = acc

        merge16(hist_v)

        pltpu.sync_copy(merged_v, shared.at[pl.ds(sid * nb, nb)])
        plsc.subcore_barrier()

        @pl.when(sid == 0)
        def _():
            pltpu.sync_copy(shared, hist_v)
            merge16(hist_v)
            pltpu.sync_copy(merged_v, hist_out.at[cid])

    return functools.partial(
        pl.kernel,
        mesh=mesh,
        compiler_params=pltpu.CompilerParams(needs_layout_passes=False),
        out_type=jax.ShapeDtypeStruct((2, nb), jnp.int32),
        scratch_types=[
            pltpu.VMEM((_CH,), jnp.float32),
            pltpu.VMEM((_CH,), jnp.float32),
            pltpu.VMEM((_CH,), jnp.float32),
            pltpu.VMEM((_CH,), jnp.float32),
            pltpu.VMEM((16 * nb,), jnp.int32),
            pltpu.VMEM((nb,), jnp.int32),
            pltpu.VMEM((_L,), jnp.int32),
            pltpu.VMEM_SHARED((16 * nb,), jnp.int32),
            pltpu.SemaphoreType.DMA,
            pltpu.SemaphoreType.DMA,
        ],
    )(body)


_sc_pass1 = _make_sc_pass(20, 12, False)
_sc_pass2 = _make_sc_pass(8, 12, True)


def _pick(hist2, kk):
    h = hist2[0] + hist2[1]
    cum = jnp.cumsum(h)
    b = jnp.argmax(cum >= kk).astype(jnp.int32)
    below = cum[b] - h[b]
    return b, below


def kernel(output, target):
    z12 = jnp.zeros((16 * 4096,), jnp.int32)

    h1 = _sc_pass1(output, target, jnp.zeros((16,), jnp.int32), z12)
    b1, below1 = _pick(h1, jnp.int32(_K))
    kk2 = jnp.int32(_K) - below1

    h2 = _sc_pass2(output, target, jnp.full((16,), b1, jnp.int32), z12)
    b2, _ = _pick(h2, kk2)

    u_tau = (b1 << 20) | (b2 << 8)
    tau_key = u_tau ^ jnp.int32(-2147483648)

    s, cnt = _tail_pass(output, target, tau_key.reshape(1, 1))

    uu = tau_key ^ ((tau_key >> 31) & jnp.int32(0x7FFFFFFF))
    tau_f = lax.bitcast_convert_type(uu, jnp.float32)
    sp_tau = jnp.log(jnp.exp(tau_f) + 1.0)

    total = s[0, 0] + (jnp.float32(_K) - cnt[0, 0].astype(jnp.float32)) * sp_tau
    return total / jnp.float32(_N)


# speedup vs baseline: 75.2709x; 1.1080x over previous
import functools

import jax
import jax.numpy as jnp
from jax import lax
from jax.experimental import pallas as pl
from jax.experimental.pallas import tpu as pltpu
from jax.experimental.pallas import tpu_sc as plsc

_N = 4194304
_K = _N // 2
_R = 4096
_C = 1024
_BR = 128
_NW = 32
_PER_W = _N // _NW
_CH = 8192
_U = 8
_L = 16


_TB = _N // 32


def _tail_body(o_ref, g_ref, tau_ref, s_ref, cnt_ref, facc_ref, cacc_ref):
    i = pl.program_id(0)
    t = o_ref[...] * g_ref[...]
    u = lax.bitcast_convert_type(t, jnp.int32)
    key = u ^ ((u >> 31) & jnp.int32(0x7FFFFFFF))
    m = key < tau_ref[0, 0]
    sp_neg = jnp.log(jnp.exp(-t) + 1.0)
    val = sp_neg + jnp.where(m, t + sp_neg, jnp.float32(0.0))
    bsum = jnp.sum(val)
    bcnt = jnp.sum(m.astype(jnp.int32))

    @pl.when(i == 0)
    def _():
        facc_ref[0, 0] = jnp.float32(0.0)
        cacc_ref[0, 0] = jnp.int32(0)

    facc_ref[0, 0] += bsum
    cacc_ref[0, 0] += bcnt

    @pl.when(i == pl.num_programs(0) - 1)
    def _():
        s_ref[0, 0] = facc_ref[0, 0]
        cnt_ref[0, 0] = cacc_ref[0, 0]


_tail_pass = pl.pallas_call(
    _tail_body,
    grid=(_N // _TB,),
    in_specs=[
        pl.BlockSpec((_TB,), lambda i: (i,)),
        pl.BlockSpec((_TB,), lambda i: (i,)),
        pl.BlockSpec(memory_space=pltpu.SMEM),
    ],
    out_specs=[
        pl.BlockSpec(memory_space=pltpu.SMEM),
        pl.BlockSpec(memory_space=pltpu.SMEM),
    ],
    out_shape=[
        jax.ShapeDtypeStruct((1, 1), jnp.float32),
        jax.ShapeDtypeStruct((1, 1), jnp.int32),
    ],
    scratch_shapes=[
        pltpu.SMEM((1, 1), jnp.float32),
        pltpu.SMEM((1, 1), jnp.int32),
    ],
)


def _make_sc_pass(shift, nbits, match):
    nb = 1 << nbits
    mesh = plsc.VectorSubcoreMesh(core_axis_name="c", subcore_axis_name="s")

    def body(o_hbm, g_hbm, *rest):
        if match:
            (h1_hbm, hist_out, o_a, g_a, o_b, g_b,
             hist_v, merged_v, h1a_v, h1b_v, shared, sem_a, sem_b) = rest
        else:
            (hist_out, o_a, g_a, o_b, g_b,
             hist_v, merged_v, shared, sem_a, sem_b) = rest
        cid = lax.axis_index("c")
        sid = lax.axis_index("s")
        wid = cid * 16 + sid
        base = wid * _PER_W

        @functools.partial(plsc.parallel_loop, 0, nb, unroll=8)
        def _(z):
            hist_v[pl.ds(z * _L, _L)] = jnp.zeros((_L,), jnp.int32)

        if match:
            pltpu.sync_copy(h1_hbm.at[0], h1a_v)
            pltpu.sync_copy(h1_hbm.at[1], h1b_v)

            def scan_body(c, carry):
                tot, bcnt = carry
                v = h1a_v[pl.ds(c * _L, _L)] + h1b_v[pl.ds(c * _L, _L)]
                pc = plsc.cumsum(v) + tot
                mlt = pc < jnp.int32(_K)
                bcnt = bcnt + plsc.all_reduce_population_count(mlt)
                tot = tot + jnp.sum(v)
                return tot, bcnt

            _, pref = lax.fori_loop(
                0, nb // _L, scan_body,
                (jnp.int32(0), jnp.zeros((_L,), jnp.int32)))
        else:
            pref = None
        lane_off = lax.iota(jnp.int32, _L) * nb
        ones = jnp.ones((_L,), jnp.int32)
        nch = _PER_W // _CH

        def start(ci, ob, gb, sem):
            pltpu.async_copy(o_hbm.at[pl.ds(base + ci * _CH, _CH)], ob, sem)
            pltpu.async_copy(g_hbm.at[pl.ds(base + ci * _CH, _CH)], gb, sem)

        def drain(ob, gb, sem):
            pltpu.make_async_copy(o_hbm.at[pl.ds(0, _CH)], ob, sem).wait()
            pltpu.make_async_copy(o_hbm.at[pl.ds(0, _CH)], gb, sem).wait()

        def process(ob, gb):
            @functools.partial(plsc.parallel_loop, 0, _CH // _L, unroll=_U)
            def _(vi):
                t = ob[pl.ds(vi * _L, _L)] * gb[pl.ds(vi * _L, _L)]
                u = lax.bitcast_convert_type(t, jnp.int32)
                uk = (u ^ ((u >> 31) & jnp.int32(0x7FFFFFFF))
                      ) ^ jnp.int32(-2147483648)
                bucket = lax.shift_right_logical(uk, shift) & jnp.int32(nb - 1)
                idx = bucket + lane_off
                if match:
                    pm = lax.shift_right_logical(uk, shift + nbits) == pref
                    plsc.addupdate_scatter(hist_v, [idx], ones, mask=pm)
                else:
                    plsc.addupdate_scatter(hist_v, [idx], ones)

        start(0, o_a, g_a, sem_a)

        def pair_body(p, carry):
            c0 = 2 * p
            start(c0 + 1, o_b, g_b, sem_b)
            drain(o_a, g_a, sem_a)
            process(o_a, g_a)

            @pl.when(c0 + 2 < nch)
            def _():
                start(c0 + 2, o_a, g_a, sem_a)

            drain(o_b, g_b, sem_b)
            process(o_b, g_b)
            return carry

        lax.fori_loop(0, nch // 2, pair_body, 0)

        def merge16(src_v):
            @functools.partial(plsc.parallel_loop, 0, nb // _L, unroll=4)
            def _(c):
                acc = src_v[pl.ds(c * _L, _L)]
                for r in range(1, 16):
                    acc = acc + src_v[pl.ds(r * nb + c * _L, _L)]
                merged_v[pl.ds(c * _L, _L)] = acc

        merge16(hist_v)

        pltpu.sync_copy(merged_v, shared.at[pl.ds(sid * nb, nb)])
        plsc.subcore_barrier()

        @pl.when(sid == 0)
        def _():
            pltpu.sync_copy(shared, hist_v)
            merge16(hist_v)
            pltpu.sync_copy(merged_v, hist_out.at[cid])

    scratch = [
        pltpu.VMEM((_CH,), jnp.float32),
        pltpu.VMEM((_CH,), jnp.float32),
        pltpu.VMEM((_CH,), jnp.float32),
        pltpu.VMEM((_CH,), jnp.float32),
        pltpu.VMEM((16 * nb,), jnp.int32),
        pltpu.VMEM((nb,), jnp.int32),
    ]
    if match:
        scratch += [pltpu.VMEM((nb,), jnp.int32), pltpu.VMEM((nb,), jnp.int32)]
    scratch += [
        pltpu.VMEM_SHARED((16 * nb,), jnp.int32),
        pltpu.SemaphoreType.DMA,
        pltpu.SemaphoreType.DMA,
    ]
    return functools.partial(
        pl.kernel,
        mesh=mesh,
        compiler_params=pltpu.CompilerParams(needs_layout_passes=False),
        out_type=jax.ShapeDtypeStruct((2, nb), jnp.int32),
        scratch_types=scratch,
    )(body)


_sc_pass1 = _make_sc_pass(20, 12, False)
_sc_pass2 = _make_sc_pass(8, 12, True)


def _pick(hist2, kk):
    h = hist2[0] + hist2[1]
    cum = jnp.cumsum(h)
    b = jnp.argmax(cum >= kk).astype(jnp.int32)
    below = cum[b] - h[b]
    return b, below


def kernel(output, target):
    h1 = _sc_pass1(output, target)
    h2 = _sc_pass2(output, target, h1)

    b1, below1 = _pick(h1, jnp.int32(_K))
    kk2 = jnp.int32(_K) - below1
    b2, _ = _pick(h2, kk2)

    u_tau = (b1 << 20) | (b2 << 8)
    tau_key = u_tau ^ jnp.int32(-2147483648)

    s, cnt = _tail_pass(output, target, tau_key.reshape(1, 1))

    uu = tau_key ^ ((tau_key >> 31) & jnp.int32(0x7FFFFFFF))
    tau_f = lax.bitcast_convert_type(uu, jnp.float32)
    sp_tau = jnp.log(jnp.exp(tau_f) + 1.0)

    total = s[0, 0] + (jnp.float32(_K) - cnt[0, 0].astype(jnp.float32)) * sp_tau
    return total / jnp.float32(_N)
